# Initial kernel scaffold; baseline (speedup 1.0000x reference)
import functools

import jax
import jax.numpy as jnp
from jax import lax
from jax.experimental import pallas as pl
from jax.experimental.pallas import tpu as pltpu
from jax.experimental.pallas import tpu_sc as plsc

N = 10000
E = 320000
D = 128
H = 128

NC = 2
NS = 16
NW = NC * NS
EPT = E // NW
B = 125
NB = EPT // B
RPT = N // NS
RB = RPT // B

W0 = 144


def _make_sc_agg(W):
  mesh = plsc.VectorSubcoreMesh(core_axis_name="c", subcore_axis_name="s")

  @functools.partial(
      pl.kernel,
      out_type=jax.ShapeDtypeStruct((NC, N, W), jnp.float32),
      mesh=mesh,
      scratch_types=[
          pltpu.VMEM((NB, B), jnp.int32),
          pltpu.VMEM((NB, B), jnp.int32),
          pltpu.VMEM((B, W), jnp.float32),
          pltpu.VMEM((B, W), jnp.float32),
          pltpu.VMEM_SHARED((N, W), jnp.float32),
          pltpu.SemaphoreType.DMA,
          pltpu.SemaphoreType.DMA,
      ],
  )
  def sc_agg(h_hbm, edge_hbm, out_hbm, srcv, dstv, buf0, buf1, acc, g0, g1):
    c = lax.axis_index("c")
    s = lax.axis_index("s")
    w = c * NS + s

    pltpu.sync_copy(edge_hbm.at[0, w], srcv)
    pltpu.sync_copy(edge_hbm.at[1, w], dstv)

    zeros16 = jnp.zeros((16,), jnp.float32)

    @pl.loop(0, B)
    def _(i):
      @pl.loop(0, W, step=16)
      def _(j):
        buf0[i, pl.ds(j, 16)] = zeros16

    @pl.loop(0, RB)
    def _(k):
      pltpu.sync_copy(buf0, acc.at[pl.ds(s * RPT + k * B, B)])

    plsc.subcore_barrier()

    pltpu.async_copy(h_hbm.at[srcv.at[0]], buf0, g0)

    @pl.loop(0, NB, step=2)
    def _(g):
      pltpu.async_copy(h_hbm.at[srcv.at[g + 1]], buf1, g1)
      pltpu.make_async_copy(h_hbm.at[srcv.at[g]], buf0, g0).wait()
      pltpu.sync_copy(buf0, acc.at[dstv.at[g]], add=True)

      @pl.when(g + 2 < NB)
      def _():
        pltpu.async_copy(h_hbm.at[srcv.at[g + 2]], buf0, g0)

      pltpu.make_async_copy(h_hbm.at[srcv.at[g + 1]], buf1, g1).wait()
      pltpu.sync_copy(buf1, acc.at[dstv.at[g + 1]], add=True)

    plsc.subcore_barrier()

    @pl.loop(0, RB)
    def _(k):
      pltpu.sync_copy(acc.at[pl.ds(s * RPT + k * B, B)],
                      out_hbm.at[c, pl.ds(s * RPT + k * B, B)])

  return sc_agg


_sc_agg_w0 = _make_sc_agg(W0)
_sc_agg_h = _make_sc_agg(H)

BN = 1000


def _dense_body(aggA, aggB, degA, degB, h, wl, wr, b, o):
  deg = degA[:] + degB[:]
  inv = 1.0 / jnp.maximum(deg, 1.0)
  agg = (aggA[:] + aggB[:]) * inv[:, None]
  o[:] = jax.nn.relu(
      jnp.dot(agg, wl[:], preferred_element_type=jnp.float32,
              precision=lax.Precision.HIGHEST)
      + jnp.dot(h[:], wr[:], preferred_element_type=jnp.float32,
                precision=lax.Precision.HIGHEST)
      + b[:][None, :])


_row_spec = pl.BlockSpec((BN, H), lambda i: (i, 0))
_deg_spec = pl.BlockSpec((BN,), lambda i: (i,))
_w_spec = pl.BlockSpec((H, H), lambda i: (0, 0))
_b_spec = pl.BlockSpec((H,), lambda i: (0,))


def _dense_layer(aggA, aggB, degA, degB, h, wl, wr, b):
  return pl.pallas_call(
      _dense_body,
      grid=(N // BN,),
      in_specs=[_row_spec, _row_spec, _deg_spec, _deg_spec, _row_spec,
                _w_spec, _w_spec, _b_spec],
      out_specs=_row_spec,
      out_shape=jax.ShapeDtypeStruct((N, H), jnp.float32),
  )(aggA, aggB, degA, degB, h, wl, wr, b)


def _final_body(aggA, aggB, degA, degB, h2, h1, wl, wr, b, wc, bc, o):
  deg = degA[:] + degB[:]
  inv = 1.0 / jnp.maximum(deg, 1.0)
  agg = (aggA[:] + aggB[:]) * inv[:, None]
  h3 = jax.nn.relu(
      jnp.dot(agg, wl[:], preferred_element_type=jnp.float32,
              precision=lax.Precision.HIGHEST)
      + jnp.dot(h2[:], wr[:], preferred_element_type=jnp.float32,
                precision=lax.Precision.HIGHEST)
      + b[:][None, :])
  out = (jnp.dot(h1[:], wc[0:H, :], preferred_element_type=jnp.float32,
                 precision=lax.Precision.HIGHEST)
         + jnp.dot(h2[:], wc[H:2 * H, :], preferred_element_type=jnp.float32,
                   precision=lax.Precision.HIGHEST)
         + jnp.dot(h3, wc[2 * H:3 * H, :], preferred_element_type=jnp.float32,
                   precision=lax.Precision.HIGHEST))
  o[:] = out[:, 0] + bc[0]


def _final_layer(aggA, aggB, degA, degB, h2, h1, wl, wr, b, wc, bc):
  return pl.pallas_call(
      _final_body,
      grid=(N // BN,),
      in_specs=[_row_spec, _row_spec, _deg_spec, _deg_spec, _row_spec,
                _row_spec, _w_spec, _w_spec, _b_spec,
                pl.BlockSpec((3 * H, 1), lambda i: (0, 0)),
                pl.BlockSpec((1,), lambda i: (0,))],
      out_specs=pl.BlockSpec((BN,), lambda i: (i,)),
      out_shape=jax.ShapeDtypeStruct((N,), jnp.float32),
  )(aggA, aggB, degA, degB, h2, h1, wl, wr, b, wc, bc)


def kernel(x, edge_index, W_l0, W_r0, b0, W_l1, W_r1, b1, W_l2, W_r2, b2,
           W_c, b_c):
  edges = edge_index.reshape(2, NW, NB, B)
  xpad = jnp.concatenate(
      [x, jnp.ones((N, 1), jnp.float32), jnp.zeros((N, W0 - D - 1),
                                                   jnp.float32)], axis=1)

  p0 = _sc_agg_w0(xpad, edges)
  aggA0 = p0[0, :, :H]
  aggB0 = p0[1, :, :H]
  degA = p0[0, :, H]
  degB = p0[1, :, H]

  h1 = _dense_layer(aggA0, aggB0, degA, degB, x, W_l0, W_r0, b0)
  p1 = _sc_agg_h(h1, edges)
  h2 = _dense_layer(p1[0], p1[1], degA, degB, h1, W_l1, W_r1, b1)
  p2 = _sc_agg_h(h2, edges)
  out = _final_layer(p2[0], p2[1], degA, degB, h2, h1, W_l2, W_r2, b2,
                     W_c, b_c)
  return out


# speedup vs baseline: 3.2670x; 3.2670x over previous
import functools

import jax
import jax.numpy as jnp
from jax import lax
from jax.experimental import pallas as pl
from jax.experimental.pallas import tpu as pltpu
from jax.experimental.pallas import tpu_sc as plsc

N = 10000
E = 320000
D = 128
H = 128

NC = 2
NS = 16
NW = NC * NS
B = 128
NB = 80
E_PAD = NW * NB * B
DUMMY = N + 4
ACC_N = N + 16

RPT = N // NS
RC = 125
RB = RPT // RC

W0 = 144


def _make_sc_agg(W):
  mesh = plsc.VectorSubcoreMesh(core_axis_name="c", subcore_axis_name="s")

  @functools.partial(
      pl.kernel,
      out_type=jax.ShapeDtypeStruct((NC, N, W), jnp.float32),
      mesh=mesh,
      scratch_types=[
          pltpu.VMEM((B,), jnp.int32),
          pltpu.VMEM((B,), jnp.int32),
          pltpu.VMEM((B,), jnp.int32),
          pltpu.VMEM((B,), jnp.int32),
          pltpu.VMEM((B, W), jnp.float32),
          pltpu.VMEM((B, W), jnp.float32),
          pltpu.VMEM_SHARED((ACC_N, W), jnp.float32),
          pltpu.SemaphoreType.DMA,
          pltpu.SemaphoreType.DMA,
          pltpu.SemaphoreType.DMA,
          pltpu.SemaphoreType.DMA,
          pltpu.SemaphoreType.DMA,
          pltpu.SemaphoreType.DMA,
      ],
      compiler_params=pltpu.CompilerParams(use_tc_tiling_on_sc=False),
  )
  def sc_agg(h_hbm, edge_hbm, out_hbm, srcv0, srcv1, dstv0, dstv1,
             buf0, buf1, acc, g0, g1, is0, is1, id0, id1):
    c = lax.axis_index("c")
    s = lax.axis_index("s")
    w = c * NS + s

    zeros16 = jnp.zeros((16,), jnp.float32)

    @pl.loop(0, B)
    def _(i):
      @pl.loop(0, W, step=16)
      def _(j):
        buf0[i, pl.ds(j, 16)] = zeros16

    @pl.loop(0, RB)
    def _(k):
      pltpu.sync_copy(buf0.at[pl.ds(0, RC)],
                      acc.at[pl.ds(s * RPT + k * RC, RC)])

    @pl.when(s == 0)
    def _():
      pltpu.sync_copy(buf0.at[pl.ds(0, ACC_N - N)], acc.at[pl.ds(N, ACC_N - N)])

    pltpu.sync_copy(edge_hbm.at[0, w, 0], srcv0)
    pltpu.sync_copy(edge_hbm.at[1, w, 0], dstv0)
    pltpu.async_copy(h_hbm.at[srcv0], buf0, g0)
    pltpu.async_copy(edge_hbm.at[0, w, 1], srcv1, is1)
    pltpu.async_copy(edge_hbm.at[1, w, 1], dstv1, id1)

    plsc.subcore_barrier()

    @pl.loop(0, NB, step=2)
    def _(g):
      pltpu.make_async_copy(edge_hbm.at[0, w, 0], srcv1, is1).wait()
      pltpu.make_async_copy(edge_hbm.at[1, w, 0], dstv1, id1).wait()
      pltpu.async_copy(h_hbm.at[srcv1], buf1, g1)

      pltpu.make_async_copy(h_hbm.at[srcv0], buf0, g0).wait()
      pltpu.sync_copy(buf0, acc.at[dstv0], add=True)

      @pl.when(g + 2 < NB)
      def _():
        pltpu.async_copy(edge_hbm.at[0, w, g + 2], srcv0, is0)
        pltpu.async_copy(edge_hbm.at[1, w, g + 2], dstv0, id0)
        pltpu.make_async_copy(edge_hbm.at[0, w, 0], srcv0, is0).wait()
        pltpu.make_async_copy(edge_hbm.at[1, w, 0], dstv0, id0).wait()
        pltpu.async_copy(h_hbm.at[srcv0], buf0, g0)

      pltpu.make_async_copy(h_hbm.at[srcv1], buf1, g1).wait()
      pltpu.sync_copy(buf1, acc.at[dstv1], add=True)

      @pl.when(g + 3 < NB)
      def _():
        pltpu.async_copy(edge_hbm.at[0, w, g + 3], srcv1, is1)
        pltpu.async_copy(edge_hbm.at[1, w, g + 3], dstv1, id1)

    plsc.subcore_barrier()

    @pl.loop(0, RB)
    def _(k):
      pltpu.sync_copy(acc.at[pl.ds(s * RPT + k * RC, RC)],
                      out_hbm.at[c, pl.ds(s * RPT + k * RC, RC)])

  return sc_agg


_sc_agg_w0 = _make_sc_agg(W0)
_sc_agg_h = _make_sc_agg(H)

BN = 1000


def _dense_body(aggA, aggB, degA, degB, h, wl, wr, b, o):
  deg = degA[:] + degB[:]
  inv = 1.0 / jnp.maximum(deg, 1.0)
  agg = (aggA[:] + aggB[:]) * inv
  o[:] = jax.nn.relu(
      jnp.dot(agg, wl[:], preferred_element_type=jnp.float32,
              precision=lax.Precision.HIGHEST)
      + jnp.dot(h[:], wr[:], preferred_element_type=jnp.float32,
                precision=lax.Precision.HIGHEST)
      + b[:][None, :])


_row_spec = pl.BlockSpec((BN, H), lambda i: (i, 0))
_deg_spec = pl.BlockSpec((BN, 1), lambda i: (i, 0))
_w_spec = pl.BlockSpec((H, H), lambda i: (0, 0))
_b_spec = pl.BlockSpec((H,), lambda i: (0,))


def _dense_layer(aggA, aggB, degA, degB, h, wl, wr, b):
  return pl.pallas_call(
      _dense_body,
      grid=(N // BN,),
      in_specs=[_row_spec, _row_spec, _deg_spec, _deg_spec, _row_spec,
                _w_spec, _w_spec, _b_spec],
      out_specs=_row_spec,
      out_shape=jax.ShapeDtypeStruct((N, H), jnp.float32),
  )(aggA, aggB, degA, degB, h, wl, wr, b)


def _final_body(aggA, aggB, degA, degB, h2, h1, wl, wr, b, wc, bc, o):
  deg = degA[:] + degB[:]
  inv = 1.0 / jnp.maximum(deg, 1.0)
  agg = (aggA[:] + aggB[:]) * inv
  h3 = jax.nn.relu(
      jnp.dot(agg, wl[:], preferred_element_type=jnp.float32,
              precision=lax.Precision.HIGHEST)
      + jnp.dot(h2[:], wr[:], preferred_element_type=jnp.float32,
                precision=lax.Precision.HIGHEST)
      + b[:][None, :])
  out = (jnp.dot(h1[:], wc[0:H, :], preferred_element_type=jnp.float32,
                 precision=lax.Precision.HIGHEST)
         + jnp.dot(h2[:], wc[H:2 * H, :], preferred_element_type=jnp.float32,
                   precision=lax.Precision.HIGHEST)
         + jnp.dot(h3, wc[2 * H:3 * H, :], preferred_element_type=jnp.float32,
                   precision=lax.Precision.HIGHEST))
  o[:] = out + bc[0]


def _final_layer(aggA, aggB, degA, degB, h2, h1, wl, wr, b, wc, bc):
  return pl.pallas_call(
      _final_body,
      grid=(N // BN,),
      in_specs=[_row_spec, _row_spec, _deg_spec, _deg_spec, _row_spec,
                _row_spec, _w_spec, _w_spec, _b_spec,
                pl.BlockSpec((3 * H, 1), lambda i: (0, 0)),
                pl.BlockSpec((1,), lambda i: (0,))],
      out_specs=pl.BlockSpec((BN, 1), lambda i: (i, 0)),
      out_shape=jax.ShapeDtypeStruct((N, 1), jnp.float32),
  )(aggA, aggB, degA, degB, h2, h1, wl, wr, b, wc, bc)


def kernel(x, edge_index, W_l0, W_r0, b0, W_l1, W_r1, b1, W_l2, W_r2, b2,
           W_c, b_c):
  pad = E_PAD - E
  src = jnp.concatenate([edge_index[0], jnp.zeros((pad,), jnp.int32)])
  dst = jnp.concatenate([edge_index[1], jnp.full((pad,), DUMMY, jnp.int32)])
  edges = jnp.stack([src, dst]).reshape(2, NW, NB, B)
  xpad = jnp.concatenate(
      [x, jnp.ones((N, 1), jnp.float32), jnp.zeros((N, W0 - D - 1),
                                                   jnp.float32)], axis=1)

  p0 = _sc_agg_w0(xpad, edges)
  aggA0 = p0[0, :, :H]
  aggB0 = p0[1, :, :H]
  degA = p0[0, :, H:H + 1]
  degB = p0[1, :, H:H + 1]

  h1 = _dense_layer(aggA0, aggB0, degA, degB, x, W_l0, W_r0, b0)
  p1 = _sc_agg_h(h1, edges)
  h2 = _dense_layer(p1[0], p1[1], degA, degB, h1, W_l1, W_r1, b1)
  p2 = _sc_agg_h(h2, edges)
  out = _final_layer(p2[0], p2[1], degA, degB, h2, h1, W_l2, W_r2, b2,
                     W_c, b_c)
  return out[:, 0]


# speedup vs baseline: 3.2687x; 1.0005x over previous
import functools

import jax
import jax.numpy as jnp
from jax import lax
from jax.experimental import pallas as pl
from jax.experimental.pallas import tpu as pltpu
from jax.experimental.pallas import tpu_sc as plsc

N = 10000
E = 320000
D = 128
H = 128

NC = 2
NS = 16
NW = NC * NS
B = 128
NB = 80
E_PAD = NW * NB * B
DUMMY = N + 4
ACC_N = N + 16

RPT = N // NS
RC = 125
RB = RPT // RC

W0 = 144


def _make_sc_agg(W):
  mesh = plsc.VectorSubcoreMesh(core_axis_name="c", subcore_axis_name="s")

  @functools.partial(
      pl.kernel,
      out_type=jax.ShapeDtypeStruct((NC, N, W), jnp.float32),
      mesh=mesh,
      scratch_types=[
          pltpu.VMEM((4, B), jnp.int32),
          pltpu.VMEM((4, B), jnp.int32),
          pltpu.VMEM((B, W), jnp.float32),
          pltpu.VMEM((B, W), jnp.float32),
          pltpu.VMEM_SHARED((ACC_N, W), jnp.float32),
          pltpu.SemaphoreType.DMA,
          pltpu.SemaphoreType.DMA,
          pltpu.SemaphoreType.DMA,
          pltpu.SemaphoreType.DMA,
          pltpu.SemaphoreType.DMA,
          pltpu.SemaphoreType.DMA,
      ],
      compiler_params=pltpu.CompilerParams(use_tc_tiling_on_sc=False),
  )
  def sc_agg(h_hbm, edge_hbm, out_hbm, srcr, dstr, buf0, buf1, acc,
             g0, g1, i0, i1, i2, i3):
    c = lax.axis_index("c")
    s = lax.axis_index("s")
    w = c * NS + s
    bufs = (buf0, buf1)
    gsem = (g0, g1)
    isem = (i0, i1, i2, i3)

    zeros16 = jnp.zeros((16,), jnp.float32)

    @pl.loop(0, B)
    def _(i):
      @pl.loop(0, W, step=16)
      def _(j):
        buf0[i, pl.ds(j, 16)] = zeros16

    @pl.loop(0, RB)
    def _(k):
      pltpu.sync_copy(buf0.at[pl.ds(0, RC)],
                      acc.at[pl.ds(s * RPT + k * RC, RC)])

    @pl.when(s == 0)
    def _():
      pltpu.sync_copy(buf0.at[pl.ds(0, ACC_N - N)], acc.at[pl.ds(N, ACC_N - N)])

    pltpu.sync_copy(edge_hbm.at[0, w, 0], srcr.at[0])
    pltpu.sync_copy(edge_hbm.at[1, w, 0], dstr.at[0])
    pltpu.sync_copy(edge_hbm.at[0, w, 1], srcr.at[1])
    pltpu.sync_copy(edge_hbm.at[1, w, 1], dstr.at[1])
    pltpu.async_copy(edge_hbm.at[0, w, 2], srcr.at[2], i2)
    pltpu.async_copy(edge_hbm.at[1, w, 2], dstr.at[2], i2)
    pltpu.async_copy(edge_hbm.at[0, w, 3], srcr.at[3], i3)
    pltpu.async_copy(edge_hbm.at[1, w, 3], dstr.at[3], i3)
    pltpu.async_copy(h_hbm.at[srcr.at[0]], buf0, g0)
    pltpu.async_copy(h_hbm.at[srcr.at[1]], buf1, g1)

    plsc.subcore_barrier()

    @pl.loop(0, NB, step=4)
    def _(g):
      for k in range(4):
        p = k % 2
        k2 = (k + 2) % 4
        b = g + k
        pltpu.make_async_copy(h_hbm.at[srcr.at[k]], bufs[p], gsem[p]).wait()
        pltpu.sync_copy(bufs[p], acc.at[dstr.at[k]], add=True)

        @pl.when(b + 2 < NB)
        def _():
          pltpu.make_async_copy(edge_hbm.at[0, w, 0], srcr.at[k2],
                                isem[k2]).wait()
          pltpu.make_async_copy(edge_hbm.at[1, w, 0], dstr.at[k2],
                                isem[k2]).wait()
          pltpu.async_copy(h_hbm.at[srcr.at[k2]], bufs[p], gsem[p])

        @pl.when(b + 4 < NB)
        def _():
          pltpu.async_copy(edge_hbm.at[0, w, b + 4], srcr.at[k], isem[k])
          pltpu.async_copy(edge_hbm.at[1, w, b + 4], dstr.at[k], isem[k])

    plsc.subcore_barrier()

    @pl.loop(0, RB)
    def _(k):
      pltpu.sync_copy(acc.at[pl.ds(s * RPT + k * RC, RC)],
                      out_hbm.at[c, pl.ds(s * RPT + k * RC, RC)])

  return sc_agg


_sc_agg_w0 = _make_sc_agg(W0)
_sc_agg_h = _make_sc_agg(H)

BN = 1000


def _dense_body(aggA, aggB, degA, degB, h, wl, wr, b, o):
  deg = degA[:] + degB[:]
  inv = 1.0 / jnp.maximum(deg, 1.0)
  agg = (aggA[:] + aggB[:]) * inv
  o[:] = jax.nn.relu(
      jnp.dot(agg, wl[:], preferred_element_type=jnp.float32,
              precision=lax.Precision.HIGHEST)
      + jnp.dot(h[:], wr[:], preferred_element_type=jnp.float32,
                precision=lax.Precision.HIGHEST)
      + b[:][None, :])


_row_spec = pl.BlockSpec((BN, H), lambda i: (i, 0))
_deg_spec = pl.BlockSpec((BN, 1), lambda i: (i, 0))
_w_spec = pl.BlockSpec((H, H), lambda i: (0, 0))
_b_spec = pl.BlockSpec((H,), lambda i: (0,))


def _dense_layer(aggA, aggB, degA, degB, h, wl, wr, b):
  return pl.pallas_call(
      _dense_body,
      grid=(N // BN,),
      in_specs=[_row_spec, _row_spec, _deg_spec, _deg_spec, _row_spec,
                _w_spec, _w_spec, _b_spec],
      out_specs=_row_spec,
      out_shape=jax.ShapeDtypeStruct((N, H), jnp.float32),
  )(aggA, aggB, degA, degB, h, wl, wr, b)


def _final_body(aggA, aggB, degA, degB, h2, h1, wl, wr, b, wc, bc, o):
  deg = degA[:] + degB[:]
  inv = 1.0 / jnp.maximum(deg, 1.0)
  agg = (aggA[:] + aggB[:]) * inv
  h3 = jax.nn.relu(
      jnp.dot(agg, wl[:], preferred_element_type=jnp.float32,
              precision=lax.Precision.HIGHEST)
      + jnp.dot(h2[:], wr[:], preferred_element_type=jnp.float32,
                precision=lax.Precision.HIGHEST)
      + b[:][None, :])
  out = (jnp.dot(h1[:], wc[0:H, :], preferred_element_type=jnp.float32,
                 precision=lax.Precision.HIGHEST)
         + jnp.dot(h2[:], wc[H:2 * H, :], preferred_element_type=jnp.float32,
                   precision=lax.Precision.HIGHEST)
         + jnp.dot(h3, wc[2 * H:3 * H, :], preferred_element_type=jnp.float32,
                   precision=lax.Precision.HIGHEST))
  o[:] = out + bc[0]


def _final_layer(aggA, aggB, degA, degB, h2, h1, wl, wr, b, wc, bc):
  return pl.pallas_call(
      _final_body,
      grid=(N // BN,),
      in_specs=[_row_spec, _row_spec, _deg_spec, _deg_spec, _row_spec,
                _row_spec, _w_spec, _w_spec, _b_spec,
                pl.BlockSpec((3 * H, 1), lambda i: (0, 0)),
                pl.BlockSpec((1,), lambda i: (0,))],
      out_specs=pl.BlockSpec((BN, 1), lambda i: (i, 0)),
      out_shape=jax.ShapeDtypeStruct((N, 1), jnp.float32),
  )(aggA, aggB, degA, degB, h2, h1, wl, wr, b, wc, bc)


def kernel(x, edge_index, W_l0, W_r0, b0, W_l1, W_r1, b1, W_l2, W_r2, b2,
           W_c, b_c):
  pad = E_PAD - E
  src = jnp.concatenate([edge_index[0], jnp.zeros((pad,), jnp.int32)])
  dst = jnp.concatenate([edge_index[1], jnp.full((pad,), DUMMY, jnp.int32)])
  edges = jnp.stack([src, dst]).reshape(2, NW, NB, B)
  xpad = jnp.concatenate(
      [x, jnp.ones((N, 1), jnp.float32), jnp.zeros((N, W0 - D - 1),
                                                   jnp.float32)], axis=1)

  p0 = _sc_agg_w0(xpad, edges)
  aggA0 = p0[0, :, :H]
  aggB0 = p0[1, :, :H]
  degA = p0[0, :, H:H + 1]
  degB = p0[1, :, H:H + 1]

  h1 = _dense_layer(aggA0, aggB0, degA, degB, x, W_l0, W_r0, b0)
  p1 = _sc_agg_h(h1, edges)
  h2 = _dense_layer(p1[0], p1[1], degA, degB, h1, W_l1, W_r1, b1)
  p2 = _sc_agg_h(h2, edges)
  out = _final_layer(p2[0], p2[1], degA, degB, h2, h1, W_l2, W_r2, b2,
                     W_c, b_c)
  return out[:, 0]


# speedup vs baseline: 10.4308x; 3.1911x over previous
import functools

import jax
import jax.numpy as jnp
from jax import lax
from jax.experimental import pallas as pl
from jax.experimental.pallas import tpu as pltpu
from jax.experimental.pallas import tpu_sc as plsc

N = 10000
E = 320000
D = 128
H = 128

NC = 2
NS = 16
NW = NC * NS
EPT = E // NW
B = 128
NBF = EPT // B
TAIL = EPT - NBF * B

RPT = N // NS
RC = 125
RB = RPT // RC

W0 = 144


def _make_sc_agg(W):
  mesh = plsc.VectorSubcoreMesh(core_axis_name="c", subcore_axis_name="s")

  @functools.partial(
      pl.kernel,
      out_type=jax.ShapeDtypeStruct((NC, N, W), jnp.float32),
      mesh=mesh,
      scratch_types=[
          pltpu.VMEM((4, B), jnp.int32),
          pltpu.VMEM((4, B), jnp.int32),
          pltpu.VMEM((B, W), jnp.float32),
          pltpu.VMEM((B, W), jnp.float32),
          pltpu.VMEM((TAIL,), jnp.int32),
          pltpu.VMEM((TAIL,), jnp.int32),
          pltpu.VMEM_SHARED((N, W), jnp.float32),
          pltpu.SemaphoreType.DMA,
          pltpu.SemaphoreType.DMA,
          pltpu.SemaphoreType.DMA,
          pltpu.SemaphoreType.DMA,
          pltpu.SemaphoreType.DMA,
          pltpu.SemaphoreType.DMA,
      ],
      compiler_params=pltpu.CompilerParams(use_tc_tiling_on_sc=False),
  )
  def sc_agg(h_hbm, edge_hbm, etail_hbm, out_hbm, srcr, dstr, buf0, buf1,
             srct, dstt, acc, g0, g1, i0, i1, i2, i3):
    c = lax.axis_index("c")
    s = lax.axis_index("s")
    w = c * NS + s
    bufs = (buf0, buf1)
    gsem = (g0, g1)
    isem = (i0, i1, i2, i3)

    zeros16 = jnp.zeros((16,), jnp.float32)

    @pl.loop(0, B)
    def _(i):
      @pl.loop(0, W, step=16)
      def _(j):
        buf0[i, pl.ds(j, 16)] = zeros16

    @pl.loop(0, RB)
    def _(k):
      pltpu.sync_copy(buf0.at[pl.ds(0, RC)],
                      acc.at[pl.ds(s * RPT + k * RC, RC)])

    pltpu.sync_copy(edge_hbm.at[0, w, 0], srcr.at[0])
    pltpu.sync_copy(edge_hbm.at[1, w, 0], dstr.at[0])
    pltpu.sync_copy(edge_hbm.at[0, w, 1], srcr.at[1])
    pltpu.sync_copy(edge_hbm.at[1, w, 1], dstr.at[1])
    pltpu.async_copy(edge_hbm.at[0, w, 2], srcr.at[2], i2)
    pltpu.async_copy(edge_hbm.at[1, w, 2], dstr.at[2], i2)
    pltpu.async_copy(edge_hbm.at[0, w, 3], srcr.at[3], i3)
    pltpu.async_copy(edge_hbm.at[1, w, 3], dstr.at[3], i3)
    pltpu.async_copy(h_hbm.at[srcr.at[0]], buf0, g0)
    pltpu.async_copy(h_hbm.at[srcr.at[1]], buf1, g1)

    plsc.subcore_barrier()

    @pl.loop(0, NBF - 2, step=4)
    def _(g):
      for k in range(4):
        p = k % 2
        k2 = (k + 2) % 4
        b = g + k
        pltpu.make_async_copy(h_hbm.at[srcr.at[k]], bufs[p], gsem[p]).wait()
        pltpu.sync_copy(bufs[p], acc.at[dstr.at[k]], add=True)

        @pl.when(b + 2 < NBF)
        def _():
          pltpu.make_async_copy(edge_hbm.at[0, w, 0], srcr.at[k2],
                                isem[k2]).wait()
          pltpu.make_async_copy(edge_hbm.at[1, w, 0], dstr.at[k2],
                                isem[k2]).wait()
          pltpu.async_copy(h_hbm.at[srcr.at[k2]], bufs[p], gsem[p])

        @pl.when(b + 4 < NBF)
        def _():
          pltpu.async_copy(edge_hbm.at[0, w, b + 4], srcr.at[k], isem[k])
          pltpu.async_copy(edge_hbm.at[1, w, b + 4], dstr.at[k], isem[k])

    pltpu.make_async_copy(h_hbm.at[srcr.at[0]], buf0, g0).wait()
    pltpu.sync_copy(buf0, acc.at[dstr.at[0]], add=True)
    pltpu.make_async_copy(h_hbm.at[srcr.at[1]], buf1, g1).wait()
    pltpu.sync_copy(buf1, acc.at[dstr.at[1]], add=True)

    pltpu.sync_copy(etail_hbm.at[0, w], srct)
    pltpu.sync_copy(etail_hbm.at[1, w], dstt)
    pltpu.sync_copy(h_hbm.at[srct], buf0.at[pl.ds(0, TAIL)])
    pltpu.sync_copy(buf0.at[pl.ds(0, TAIL)], acc.at[dstt], add=True)

    plsc.subcore_barrier()

    @pl.loop(0, RB)
    def _(k):
      pltpu.sync_copy(acc.at[pl.ds(s * RPT + k * RC, RC)],
                      out_hbm.at[c, pl.ds(s * RPT + k * RC, RC)])

  return sc_agg


_sc_agg_w0 = _make_sc_agg(W0)
_sc_agg_h = _make_sc_agg(H)

BN = 1000


def _dense_body(aggA, aggB, degA, degB, h, wl, wr, b, o):
  deg = degA[:] + degB[:]
  inv = 1.0 / jnp.maximum(deg, 1.0)
  agg = (aggA[:] + aggB[:]) * inv
  o[:] = jax.nn.relu(
      jnp.dot(agg, wl[:], preferred_element_type=jnp.float32,
              precision=lax.Precision.HIGHEST)
      + jnp.dot(h[:], wr[:], preferred_element_type=jnp.float32,
                precision=lax.Precision.HIGHEST)
      + b[:][None, :])


_row_spec = pl.BlockSpec((BN, H), lambda i: (i, 0))
_deg_spec = pl.BlockSpec((BN, 1), lambda i: (i, 0))
_w_spec = pl.BlockSpec((H, H), lambda i: (0, 0))
_b_spec = pl.BlockSpec((H,), lambda i: (0,))


def _dense_layer(aggA, aggB, degA, degB, h, wl, wr, b):
  return pl.pallas_call(
      _dense_body,
      grid=(N // BN,),
      in_specs=[_row_spec, _row_spec, _deg_spec, _deg_spec, _row_spec,
                _w_spec, _w_spec, _b_spec],
      out_specs=_row_spec,
      out_shape=jax.ShapeDtypeStruct((N, H), jnp.float32),
  )(aggA, aggB, degA, degB, h, wl, wr, b)


def _final_body(aggA, aggB, degA, degB, h2, h1, wl, wr, b, wc, bc, o):
  deg = degA[:] + degB[:]
  inv = 1.0 / jnp.maximum(deg, 1.0)
  agg = (aggA[:] + aggB[:]) * inv
  h3 = jax.nn.relu(
      jnp.dot(agg, wl[:], preferred_element_type=jnp.float32,
              precision=lax.Precision.HIGHEST)
      + jnp.dot(h2[:], wr[:], preferred_element_type=jnp.float32,
                precision=lax.Precision.HIGHEST)
      + b[:][None, :])
  out = (jnp.dot(h1[:], wc[0:H, :], preferred_element_type=jnp.float32,
                 precision=lax.Precision.HIGHEST)
         + jnp.dot(h2[:], wc[H:2 * H, :], preferred_element_type=jnp.float32,
                   precision=lax.Precision.HIGHEST)
         + jnp.dot(h3, wc[2 * H:3 * H, :], preferred_element_type=jnp.float32,
                   precision=lax.Precision.HIGHEST))
  o[:] = out + bc[0]


def _final_layer(aggA, aggB, degA, degB, h2, h1, wl, wr, b, wc, bc):
  return pl.pallas_call(
      _final_body,
      grid=(N // BN,),
      in_specs=[_row_spec, _row_spec, _deg_spec, _deg_spec, _row_spec,
                _row_spec, _w_spec, _w_spec, _b_spec,
                pl.BlockSpec((3 * H, 1), lambda i: (0, 0)),
                pl.BlockSpec((1,), lambda i: (0,))],
      out_specs=pl.BlockSpec((BN, 1), lambda i: (i, 0)),
      out_shape=jax.ShapeDtypeStruct((N, 1), jnp.float32),
  )(aggA, aggB, degA, degB, h2, h1, wl, wr, b, wc, bc)


def kernel(x, edge_index, W_l0, W_r0, b0, W_l1, W_r1, b1, W_l2, W_r2, b2,
           W_c, b_c):
  e = edge_index.reshape(2, NW, EPT)
  e_main = e[:, :, :NBF * B].reshape(2, NW, NBF, B)
  e_tail = e[:, :, NBF * B:]
  xpad = jnp.concatenate(
      [x, jnp.ones((N, 1), jnp.float32), jnp.zeros((N, W0 - D - 1),
                                                   jnp.float32)], axis=1)

  p0 = _sc_agg_w0(xpad, e_main, e_tail)
  aggA0 = p0[0, :, :H]
  aggB0 = p0[1, :, :H]
  degA = p0[0, :, H:H + 1]
  degB = p0[1, :, H:H + 1]

  h1 = _dense_layer(aggA0, aggB0, degA, degB, x, W_l0, W_r0, b0)
  p1 = _sc_agg_h(h1, e_main, e_tail)
  h2 = _dense_layer(p1[0], p1[1], degA, degB, h1, W_l1, W_r1, b1)
  p2 = _sc_agg_h(h2, e_main, e_tail)
  out = _final_layer(p2[0], p2[1], degA, degB, h2, h1, W_l2, W_r2, b2,
                     W_c, b_c)
  return out[:, 0]


# speedup vs baseline: 11.3851x; 1.0915x over previous
import functools

import jax
import jax.numpy as jnp
from jax import lax
from jax.experimental import pallas as pl
from jax.experimental.pallas import tpu as pltpu
from jax.experimental.pallas import tpu_sc as plsc

N = 10000
E = 320000
D = 128
H = 128

NC = 2
NS = 16
NW = NC * NS
EPT = E // NW
B = 128
NBF = EPT // B
TAIL = EPT - NBF * B

RPT = N // NS
RC = 125
RB = RPT // RC

W0 = 144


def _make_sc_agg(W):
  mesh = plsc.VectorSubcoreMesh(core_axis_name="c", subcore_axis_name="s")

  @functools.partial(
      pl.kernel,
      out_type=jax.ShapeDtypeStruct((NC, N, W), jnp.float32),
      mesh=mesh,
      scratch_types=[
          pltpu.VMEM((4, B), jnp.int32),
          pltpu.VMEM((4, B), jnp.int32),
          pltpu.VMEM((B, W), jnp.float32),
          pltpu.VMEM((B, W), jnp.float32),
          pltpu.VMEM((TAIL,), jnp.int32),
          pltpu.VMEM((TAIL,), jnp.int32),
          pltpu.VMEM_SHARED((N, W), jnp.float32),
          pltpu.SemaphoreType.DMA,
          pltpu.SemaphoreType.DMA,
          pltpu.SemaphoreType.DMA,
          pltpu.SemaphoreType.DMA,
          pltpu.SemaphoreType.DMA,
          pltpu.SemaphoreType.DMA,
      ],
      compiler_params=pltpu.CompilerParams(use_tc_tiling_on_sc=False),
  )
  def sc_agg(h_hbm, edge_hbm, etail_hbm, out_hbm, srcr, dstr, buf0, buf1,
             srct, dstt, acc, g0, g1, i0, i1, i2, i3):
    c = lax.axis_index("c")
    s = lax.axis_index("s")
    w = c * NS + s
    bufs = (buf0, buf1)
    gsem = (g0, g1)
    isem = (i0, i1, i2, i3)

    zeros16 = jnp.zeros((16,), jnp.float32)

    @pl.loop(0, B)
    def _(i):
      @pl.loop(0, W, step=16)
      def _(j):
        buf0[i, pl.ds(j, 16)] = zeros16

    @pl.loop(0, RB)
    def _(k):
      pltpu.sync_copy(buf0.at[pl.ds(0, RC)],
                      acc.at[pl.ds(s * RPT + k * RC, RC)])

    pltpu.sync_copy(edge_hbm.at[0, w, 0], srcr.at[0])
    pltpu.sync_copy(edge_hbm.at[1, w, 0], dstr.at[0])
    pltpu.sync_copy(edge_hbm.at[0, w, 1], srcr.at[1])
    pltpu.sync_copy(edge_hbm.at[1, w, 1], dstr.at[1])
    pltpu.async_copy(edge_hbm.at[0, w, 2], srcr.at[2], i2)
    pltpu.async_copy(edge_hbm.at[1, w, 2], dstr.at[2], i2)
    pltpu.async_copy(edge_hbm.at[0, w, 3], srcr.at[3], i3)
    pltpu.async_copy(edge_hbm.at[1, w, 3], dstr.at[3], i3)
    pltpu.async_copy(h_hbm.at[srcr.at[0]], buf0, g0)
    pltpu.async_copy(h_hbm.at[srcr.at[1]], buf1, g1)

    plsc.subcore_barrier()

    @pl.loop(0, NBF - 2, step=4)
    def _(g):
      for k in range(4):
        p = k % 2
        k2 = (k + 2) % 4
        b = g + k
        pltpu.make_async_copy(h_hbm.at[srcr.at[k]], bufs[p], gsem[p]).wait()
        pltpu.sync_copy(bufs[p], acc.at[dstr.at[k]], add=True)

        @pl.when(b + 2 < NBF)
        def _():
          pltpu.make_async_copy(edge_hbm.at[0, w, 0], srcr.at[k2],
                                isem[k2]).wait()
          pltpu.make_async_copy(edge_hbm.at[1, w, 0], dstr.at[k2],
                                isem[k2]).wait()
          pltpu.async_copy(h_hbm.at[srcr.at[k2]], bufs[p], gsem[p])

        @pl.when(b + 4 < NBF)
        def _():
          pltpu.async_copy(edge_hbm.at[0, w, b + 4], srcr.at[k], isem[k])
          pltpu.async_copy(edge_hbm.at[1, w, b + 4], dstr.at[k], isem[k])

    pltpu.make_async_copy(h_hbm.at[srcr.at[0]], buf0, g0).wait()
    pltpu.sync_copy(buf0, acc.at[dstr.at[0]], add=True)
    pltpu.make_async_copy(h_hbm.at[srcr.at[1]], buf1, g1).wait()
    pltpu.sync_copy(buf1, acc.at[dstr.at[1]], add=True)

    pltpu.sync_copy(etail_hbm.at[0, w], srct)
    pltpu.sync_copy(etail_hbm.at[1, w], dstt)
    pltpu.sync_copy(h_hbm.at[srct], buf0.at[pl.ds(0, TAIL)])
    pltpu.sync_copy(buf0.at[pl.ds(0, TAIL)], acc.at[dstt], add=True)

    plsc.subcore_barrier()

    @pl.loop(0, RB)
    def _(k):
      pltpu.sync_copy(acc.at[pl.ds(s * RPT + k * RC, RC)],
                      out_hbm.at[c, pl.ds(s * RPT + k * RC, RC)])

  return sc_agg


_sc_agg_w0 = _make_sc_agg(W0)
_sc_agg_h = _make_sc_agg(H)

BN = 1000


def _dense_body(aggA, aggB, degA, degB, h, wl, wr, b, o):
  deg = degA[:] + degB[:]
  inv = 1.0 / jnp.maximum(deg, 1.0)
  agg = (aggA[:] + aggB[:]) * inv
  o[:] = jax.nn.relu(
      jnp.dot(agg, wl[:], preferred_element_type=jnp.float32,
              precision=lax.Precision.DEFAULT)
      + jnp.dot(h[:], wr[:], preferred_element_type=jnp.float32,
                precision=lax.Precision.DEFAULT)
      + b[:][None, :])


_row_spec = pl.BlockSpec((BN, H), lambda i: (i, 0))
_deg_spec = pl.BlockSpec((BN, 1), lambda i: (i, 0))
_w_spec = pl.BlockSpec((H, H), lambda i: (0, 0))
_b_spec = pl.BlockSpec((H,), lambda i: (0,))


def _dense_layer(aggA, aggB, degA, degB, h, wl, wr, b):
  return pl.pallas_call(
      _dense_body,
      grid=(N // BN,),
      in_specs=[_row_spec, _row_spec, _deg_spec, _deg_spec, _row_spec,
                _w_spec, _w_spec, _b_spec],
      out_specs=_row_spec,
      out_shape=jax.ShapeDtypeStruct((N, H), jnp.float32),
  )(aggA, aggB, degA, degB, h, wl, wr, b)


def _final_body(aggA, aggB, degA, degB, h2, h1, wl, wr, b, wc, bc, o):
  deg = degA[:] + degB[:]
  inv = 1.0 / jnp.maximum(deg, 1.0)
  agg = (aggA[:] + aggB[:]) * inv
  h3 = jax.nn.relu(
      jnp.dot(agg, wl[:], preferred_element_type=jnp.float32,
              precision=lax.Precision.DEFAULT)
      + jnp.dot(h2[:], wr[:], preferred_element_type=jnp.float32,
                precision=lax.Precision.DEFAULT)
      + b[:][None, :])
  out = (jnp.dot(h1[:], wc[0:H, :], preferred_element_type=jnp.float32,
                 precision=lax.Precision.DEFAULT)
         + jnp.dot(h2[:], wc[H:2 * H, :], preferred_element_type=jnp.float32,
                   precision=lax.Precision.DEFAULT)
         + jnp.dot(h3, wc[2 * H:3 * H, :], preferred_element_type=jnp.float32,
                   precision=lax.Precision.DEFAULT))
  o[:] = out + bc[0]


def _final_layer(aggA, aggB, degA, degB, h2, h1, wl, wr, b, wc, bc):
  return pl.pallas_call(
      _final_body,
      grid=(N // BN,),
      in_specs=[_row_spec, _row_spec, _deg_spec, _deg_spec, _row_spec,
                _row_spec, _w_spec, _w_spec, _b_spec,
                pl.BlockSpec((3 * H, 1), lambda i: (0, 0)),
                pl.BlockSpec((1,), lambda i: (0,))],
      out_specs=pl.BlockSpec((BN, 1), lambda i: (i, 0)),
      out_shape=jax.ShapeDtypeStruct((N, 1), jnp.float32),
  )(aggA, aggB, degA, degB, h2, h1, wl, wr, b, wc, bc)


def kernel(x, edge_index, W_l0, W_r0, b0, W_l1, W_r1, b1, W_l2, W_r2, b2,
           W_c, b_c):
  e = edge_index.reshape(2, NW, EPT)
  e_main = e[:, :, :NBF * B].reshape(2, NW, NBF, B)
  e_tail = e[:, :, NBF * B:]
  xpad = jnp.concatenate(
      [x, jnp.ones((N, 1), jnp.float32), jnp.zeros((N, W0 - D - 1),
                                                   jnp.float32)], axis=1)

  p0 = _sc_agg_w0(xpad, e_main, e_tail)
  aggA0 = p0[0, :, :H]
  aggB0 = p0[1, :, :H]
  degA = p0[0, :, H:H + 1]
  degB = p0[1, :, H:H + 1]

  h1 = _dense_layer(aggA0, aggB0, degA, degB, x, W_l0, W_r0, b0)
  p1 = _sc_agg_h(h1, e_main, e_tail)
  h2 = _dense_layer(p1[0], p1[1], degA, degB, h1, W_l1, W_r1, b1)
  p2 = _sc_agg_h(h2, e_main, e_tail)
  out = _final_layer(p2[0], p2[1], degA, degB, h2, h1, W_l2, W_r2, b2,
                     W_c, b_c)
  return out[:, 0]


# speedup vs baseline: 12.1311x; 1.0655x over previous
import functools

import jax
import jax.numpy as jnp
from jax import lax
from jax.experimental import pallas as pl
from jax.experimental.pallas import tpu as pltpu
from jax.experimental.pallas import tpu_sc as plsc

N = 10000
E = 320000
D = 128
H = 128

NC = 2
NS = 16
NW = NC * NS
EPT = E // NW
B = 128
NBF = EPT // B
TAIL = EPT - NBF * B

RPT = N // NS
RC = 125
RB = RPT // RC

W0 = 144


def _make_sc_agg(W):
  mesh = plsc.VectorSubcoreMesh(core_axis_name="c", subcore_axis_name="s")

  @functools.partial(
      pl.kernel,
      out_type=jax.ShapeDtypeStruct((NC, N, W), jnp.float32),
      mesh=mesh,
      scratch_types=[
          pltpu.VMEM((4, B), jnp.int32),
          pltpu.VMEM((4, B), jnp.int32),
          pltpu.VMEM((B, W), jnp.float32),
          pltpu.VMEM((B, W), jnp.float32),
          pltpu.VMEM((TAIL,), jnp.int32),
          pltpu.VMEM((TAIL,), jnp.int32),
          pltpu.VMEM_SHARED((N, W), jnp.float32),
          pltpu.SemaphoreType.DMA,
          pltpu.SemaphoreType.DMA,
          pltpu.SemaphoreType.DMA,
          pltpu.SemaphoreType.DMA,
          pltpu.SemaphoreType.DMA,
          pltpu.SemaphoreType.DMA,
      ],
      compiler_params=pltpu.CompilerParams(use_tc_tiling_on_sc=False),
  )
  def sc_agg(h_hbm, edge_hbm, etail_hbm, out_hbm, srcr, dstr, buf0, buf1,
             srct, dstt, acc, g0, g1, i0, i1, i2, i3):
    c = lax.axis_index("c")
    s = lax.axis_index("s")
    w = c * NS + s
    bufs = (buf0, buf1)
    gsem = (g0, g1)
    isem = (i0, i1, i2, i3)

    zeros16 = jnp.zeros((16,), jnp.float32)

    @pl.loop(0, B)
    def _(i):
      @pl.loop(0, W, step=16)
      def _(j):
        buf0[i, pl.ds(j, 16)] = zeros16

    @pl.loop(0, RB)
    def _(k):
      pltpu.sync_copy(buf0.at[pl.ds(0, RC)],
                      acc.at[pl.ds(s * RPT + k * RC, RC)])

    pltpu.sync_copy(edge_hbm.at[0, w, 0], srcr.at[0])
    pltpu.sync_copy(edge_hbm.at[1, w, 0], dstr.at[0])
    pltpu.sync_copy(edge_hbm.at[0, w, 1], srcr.at[1])
    pltpu.sync_copy(edge_hbm.at[1, w, 1], dstr.at[1])
    pltpu.async_copy(edge_hbm.at[0, w, 2], srcr.at[2], i2)
    pltpu.async_copy(edge_hbm.at[1, w, 2], dstr.at[2], i2)
    pltpu.async_copy(edge_hbm.at[0, w, 3], srcr.at[3], i3)
    pltpu.async_copy(edge_hbm.at[1, w, 3], dstr.at[3], i3)
    pltpu.async_copy(h_hbm.at[srcr.at[0]], buf0, g0)
    pltpu.async_copy(h_hbm.at[srcr.at[1]], buf1, g1)

    plsc.subcore_barrier()

    @pl.loop(0, NBF - 2, step=4)
    def _(g):
      for k in range(4):
        p = k % 2
        k2 = (k + 2) % 4
        b = g + k
        pltpu.make_async_copy(h_hbm.at[srcr.at[k]], bufs[p], gsem[p]).wait()
        pltpu.sync_copy(bufs[p], acc.at[dstr.at[k]], add=True)

        @pl.when(b + 2 < NBF)
        def _():
          pltpu.make_async_copy(edge_hbm.at[0, w, 0], srcr.at[k2],
                                isem[k2]).wait()
          pltpu.make_async_copy(edge_hbm.at[1, w, 0], dstr.at[k2],
                                isem[k2]).wait()
          pltpu.async_copy(h_hbm.at[srcr.at[k2]], bufs[p], gsem[p])

        @pl.when(b + 4 < NBF)
        def _():
          pltpu.async_copy(edge_hbm.at[0, w, b + 4], srcr.at[k], isem[k])
          pltpu.async_copy(edge_hbm.at[1, w, b + 4], dstr.at[k], isem[k])

    pltpu.make_async_copy(h_hbm.at[srcr.at[0]], buf0, g0).wait()
    pltpu.sync_copy(buf0, acc.at[dstr.at[0]], add=True)
    pltpu.make_async_copy(h_hbm.at[srcr.at[1]], buf1, g1).wait()
    pltpu.sync_copy(buf1, acc.at[dstr.at[1]], add=True)

    pltpu.sync_copy(etail_hbm.at[0, w], srct)
    pltpu.sync_copy(etail_hbm.at[1, w], dstt)
    pltpu.sync_copy(h_hbm.at[srct], buf0.at[pl.ds(0, TAIL)])
    pltpu.sync_copy(buf0.at[pl.ds(0, TAIL)], acc.at[dstt], add=True)

    plsc.subcore_barrier()

    @pl.loop(0, RB)
    def _(k):
      pltpu.sync_copy(acc.at[pl.ds(s * RPT + k * RC, RC)],
                      out_hbm.at[c, pl.ds(s * RPT + k * RC, RC)])

  return sc_agg


_sc_agg_w0 = _make_sc_agg(W0)
_sc_agg_h = _make_sc_agg(H)

BN = 1000

_DOT = dict(preferred_element_type=jnp.float32,
            precision=lax.Precision.DEFAULT)

_row_spec = pl.BlockSpec((BN, H), lambda i: (i, 0))
_col_spec = pl.BlockSpec((BN, 1), lambda i: (i, 0))
_w_spec = pl.BlockSpec((H, H), lambda i: (0, 0))
_b_spec = pl.BlockSpec((H,), lambda i: (0,))
_wc_spec = pl.BlockSpec((H, 1), lambda i: (0, 0))


def _pA_spec(W):
  return pl.BlockSpec((1, BN, W), lambda i: (0, i, 0))


def _pB_spec(W):
  return pl.BlockSpec((1, BN, W), lambda i: (1, i, 0))


def _tca0_body(h, wr, b, r):
  r[:] = jnp.dot(h[:], wr[:], **_DOT) + b[:][None, :]


def _tca0(h, wr, b):
  return pl.pallas_call(
      _tca0_body,
      grid=(N // BN,),
      in_specs=[_row_spec, _w_spec, _b_spec],
      out_specs=_row_spec,
      out_shape=jax.ShapeDtypeStruct((N, H), jnp.float32),
  )(h, wr, b)


def _tca_body(h, wr, b, wc, r, cpart):
  r[:] = jnp.dot(h[:], wr[:], **_DOT) + b[:][None, :]
  cpart[:] = jnp.dot(h[:], wc[:], **_DOT)


def _tca(h, wr, b, wc):
  return pl.pallas_call(
      _tca_body,
      grid=(N // BN,),
      in_specs=[_row_spec, _w_spec, _b_spec, _wc_spec],
      out_specs=[_row_spec, _col_spec],
      out_shape=[jax.ShapeDtypeStruct((N, H), jnp.float32),
                 jax.ShapeDtypeStruct((N, 1), jnp.float32)],
  )(h, wr, b, wc)


def _tcb0_body(pA, pB, r, wl, h_out, dinv_out):
  deg = pA[0, :, H:H + 1] + pB[0, :, H:H + 1]
  inv = 1.0 / jnp.maximum(deg, 1.0)
  agg = (pA[0, :, :H] + pB[0, :, :H]) * inv
  h_out[:] = jax.nn.relu(jnp.dot(agg, wl[:], **_DOT) + r[:])
  dinv_out[:] = inv


def _tcb0(p, r, wl):
  return pl.pallas_call(
      _tcb0_body,
      grid=(N // BN,),
      in_specs=[_pA_spec(W0), _pB_spec(W0), _row_spec, _w_spec],
      out_specs=[_row_spec, _col_spec],
      out_shape=[jax.ShapeDtypeStruct((N, H), jnp.float32),
                 jax.ShapeDtypeStruct((N, 1), jnp.float32)],
  )(p, p, r, wl)


def _tcb1_body(pA, pB, dinv, r, wl, h_out):
  agg = (pA[0] + pB[0]) * dinv[:]
  h_out[:] = jax.nn.relu(jnp.dot(agg, wl[:], **_DOT) + r[:])


def _tcb1(p, dinv, r, wl):
  return pl.pallas_call(
      _tcb1_body,
      grid=(N // BN,),
      in_specs=[_pA_spec(H), _pB_spec(H), _col_spec, _row_spec, _w_spec],
      out_specs=_row_spec,
      out_shape=jax.ShapeDtypeStruct((N, H), jnp.float32),
  )(p, p, dinv, r, wl)


def _tcb2_body(pA, pB, dinv, r, wl, wc3, c1, c2, bc, o):
  agg = (pA[0] + pB[0]) * dinv[:]
  h3 = jax.nn.relu(jnp.dot(agg, wl[:], **_DOT) + r[:])
  o[:] = jnp.dot(h3, wc3[:], **_DOT) + c1[:] + c2[:] + bc[0]


def _tcb2(p, dinv, r, wl, wc3, c1, c2, bc):
  return pl.pallas_call(
      _tcb2_body,
      grid=(N // BN,),
      in_specs=[_pA_spec(H), _pB_spec(H), _col_spec, _row_spec, _w_spec,
                _wc_spec, _col_spec, _col_spec,
                pl.BlockSpec((1,), lambda i: (0,))],
      out_specs=_col_spec,
      out_shape=jax.ShapeDtypeStruct((N, 1), jnp.float32),
  )(p, p, dinv, r, wl, wc3, c1, c2, bc)


def kernel(x, edge_index, W_l0, W_r0, b0, W_l1, W_r1, b1, W_l2, W_r2, b2,
           W_c, b_c):
  e = edge_index.reshape(2, NW, EPT)
  e_main = e[:, :, :NBF * B].reshape(2, NW, NBF, B)
  e_tail = e[:, :, NBF * B:]
  xpad = jnp.concatenate(
      [x, jnp.ones((N, 1), jnp.float32), jnp.zeros((N, W0 - D - 1),
                                                   jnp.float32)], axis=1)

  p0 = _sc_agg_w0(xpad, e_main, e_tail)
  r0 = _tca0(x, W_r0, b0)
  h1, dinv = _tcb0(p0, r0, W_l0)

  p1 = _sc_agg_h(h1, e_main, e_tail)
  r1, c1 = _tca(h1, W_r1, b1, W_c[0:H])
  h2 = _tcb1(p1, dinv, r1, W_l1)

  p2 = _sc_agg_h(h2, e_main, e_tail)
  r2, c2 = _tca(h2, W_r2, b2, W_c[H:2 * H])
  out = _tcb2(p2, dinv, r2, W_l2, W_c[2 * H:3 * H], c1, c2, b_c)
  return out[:, 0]


# speedup vs baseline: 13.4639x; 1.1099x over previous
import functools

import jax
import jax.numpy as jnp
from jax import lax
from jax.experimental import pallas as pl
from jax.experimental.pallas import tpu as pltpu
from jax.experimental.pallas import tpu_sc as plsc

N = 10000
E = 320000
D = 128
H = 128

NC = 2
NS = 16
NW = NC * NS
EPT = E // NW
B = 128
NBF = EPT // B
TAIL = EPT - NBF * B

RPT = N // NS
RC = 125
RB = RPT // RC

def _make_sc_agg(W, with_hist):
  mesh = plsc.VectorSubcoreMesh(core_axis_name="c", subcore_axis_name="s")

  out_type = jax.ShapeDtypeStruct((NC, N, W), jnp.float32)
  if with_hist:
    out_type = (out_type, jax.ShapeDtypeStruct((NW, N), jnp.float32))

  @functools.partial(
      pl.kernel,
      out_type=out_type,
      mesh=mesh,
      scratch_types=([pltpu.VMEM((N,), jnp.float32)] if with_hist else []) + [
          pltpu.VMEM((4, B), jnp.int32),
          pltpu.VMEM((4, B), jnp.int32),
          pltpu.VMEM((B, W), jnp.float32),
          pltpu.VMEM((B, W), jnp.float32),
          pltpu.VMEM((TAIL,), jnp.int32),
          pltpu.VMEM((TAIL,), jnp.int32),
          pltpu.VMEM_SHARED((N, W), jnp.float32),
          pltpu.SemaphoreType.DMA,
          pltpu.SemaphoreType.DMA,
          pltpu.SemaphoreType.DMA,
          pltpu.SemaphoreType.DMA,
          pltpu.SemaphoreType.DMA,
          pltpu.SemaphoreType.DMA,
      ],
      compiler_params=pltpu.CompilerParams(use_tc_tiling_on_sc=False,
                                           needs_layout_passes=False),
  )
  def sc_agg(*args):
    if with_hist:
      (h_hbm, edge_hbm, etail_hbm, out_hbm, hist_hbm, hist,
       srcr, dstr, buf0, buf1, srct, dstt, acc, g0, g1, i0, i1, i2, i3) = args
    else:
      (h_hbm, edge_hbm, etail_hbm, out_hbm,
       srcr, dstr, buf0, buf1, srct, dstt, acc, g0, g1, i0, i1, i2, i3) = args
    c = lax.axis_index("c")
    s = lax.axis_index("s")
    w = c * NS + s
    bufs = (buf0, buf1)
    gsem = (g0, g1)
    isem = (i0, i1, i2, i3)

    zeros16 = jnp.zeros((16,), jnp.float32)
    ones16 = jnp.ones((16,), jnp.float32)

    def hist_update(idx_row):
      if with_hist:
        for j in range(0, B, 16):
          plsc.addupdate_scatter(hist, [idx_row[pl.ds(j, 16)]], ones16)

    @pl.loop(0, B)
    def _(i):
      @pl.loop(0, W, step=16)
      def _(j):
        buf0[i, pl.ds(j, 16)] = zeros16

    @pl.loop(0, RB)
    def _(k):
      pltpu.sync_copy(buf0.at[pl.ds(0, RC)],
                      acc.at[pl.ds(s * RPT + k * RC, RC)])

    pltpu.sync_copy(edge_hbm.at[0, w, 0], srcr.at[0])
    pltpu.sync_copy(edge_hbm.at[1, w, 0], dstr.at[0])
    pltpu.sync_copy(edge_hbm.at[0, w, 1], srcr.at[1])
    pltpu.sync_copy(edge_hbm.at[1, w, 1], dstr.at[1])
    pltpu.async_copy(edge_hbm.at[0, w, 2], srcr.at[2], i2)
    pltpu.async_copy(edge_hbm.at[1, w, 2], dstr.at[2], i2)
    pltpu.async_copy(edge_hbm.at[0, w, 3], srcr.at[3], i3)
    pltpu.async_copy(edge_hbm.at[1, w, 3], dstr.at[3], i3)
    pltpu.async_copy(h_hbm.at[srcr.at[0]], buf0, g0)
    pltpu.async_copy(h_hbm.at[srcr.at[1]], buf1, g1)

    if with_hist:
      @pl.loop(0, N, step=16)
      def _(i):
        hist[pl.ds(i, 16)] = zeros16

    plsc.subcore_barrier()

    @pl.loop(0, NBF - 2, step=4)
    def _(g):
      for k in range(4):
        p = k % 2
        k2 = (k + 2) % 4
        b = g + k
        pltpu.make_async_copy(h_hbm.at[srcr.at[k]], bufs[p], gsem[p]).wait()
        pltpu.sync_copy(bufs[p], acc.at[dstr.at[k]], add=True)
        hist_update(dstr.at[k])

        @pl.when(b + 2 < NBF)
        def _():
          pltpu.make_async_copy(edge_hbm.at[0, w, 0], srcr.at[k2],
                                isem[k2]).wait()
          pltpu.make_async_copy(edge_hbm.at[1, w, 0], dstr.at[k2],
                                isem[k2]).wait()
          pltpu.async_copy(h_hbm.at[srcr.at[k2]], bufs[p], gsem[p])

        @pl.when(b + 4 < NBF)
        def _():
          pltpu.async_copy(edge_hbm.at[0, w, b + 4], srcr.at[k], isem[k])
          pltpu.async_copy(edge_hbm.at[1, w, b + 4], dstr.at[k], isem[k])

    pltpu.make_async_copy(h_hbm.at[srcr.at[0]], buf0, g0).wait()
    pltpu.sync_copy(buf0, acc.at[dstr.at[0]], add=True)
    hist_update(dstr.at[0])
    pltpu.make_async_copy(h_hbm.at[srcr.at[1]], buf1, g1).wait()
    pltpu.sync_copy(buf1, acc.at[dstr.at[1]], add=True)
    hist_update(dstr.at[1])

    pltpu.sync_copy(etail_hbm.at[0, w], srct)
    pltpu.sync_copy(etail_hbm.at[1, w], dstt)
    pltpu.sync_copy(h_hbm.at[srct], buf0.at[pl.ds(0, TAIL)])
    pltpu.sync_copy(buf0.at[pl.ds(0, TAIL)], acc.at[dstt], add=True)
    if with_hist:
      plsc.addupdate_scatter(hist, [dstt[...]], ones16)
      pltpu.sync_copy(hist, hist_hbm.at[w])

    plsc.subcore_barrier()

    @pl.loop(0, RB)
    def _(k):
      pltpu.sync_copy(acc.at[pl.ds(s * RPT + k * RC, RC)],
                      out_hbm.at[c, pl.ds(s * RPT + k * RC, RC)])

  return sc_agg


_sc_agg_0 = _make_sc_agg(H, with_hist=True)
_sc_agg_h = _make_sc_agg(H, with_hist=False)

BN = 1000

_DOT = dict(preferred_element_type=jnp.float32,
            precision=lax.Precision.DEFAULT)

_row_spec = pl.BlockSpec((BN, H), lambda i: (i, 0))
_col_spec = pl.BlockSpec((BN, 1), lambda i: (i, 0))
_w_spec = pl.BlockSpec((H, H), lambda i: (0, 0))
_b_spec = pl.BlockSpec((H,), lambda i: (0,))
_wc_spec = pl.BlockSpec((H, 1), lambda i: (0, 0))


def _pA_spec(W):
  return pl.BlockSpec((1, BN, W), lambda i: (0, i, 0))


def _pB_spec(W):
  return pl.BlockSpec((1, BN, W), lambda i: (1, i, 0))


def _tca0_body(h, wr, b, r):
  r[:] = jnp.dot(h[:], wr[:], **_DOT) + b[:][None, :]


def _tca0(h, wr, b):
  return pl.pallas_call(
      _tca0_body,
      grid=(N // BN,),
      in_specs=[_row_spec, _w_spec, _b_spec],
      out_specs=_row_spec,
      out_shape=jax.ShapeDtypeStruct((N, H), jnp.float32),
  )(h, wr, b)


def _tca_body(h, wr, b, wc, r, cpart):
  r[:] = jnp.dot(h[:], wr[:], **_DOT) + b[:][None, :]
  cpart[:] = jnp.dot(h[:], wc[:], **_DOT)


def _tca(h, wr, b, wc):
  return pl.pallas_call(
      _tca_body,
      grid=(N // BN,),
      in_specs=[_row_spec, _w_spec, _b_spec, _wc_spec],
      out_specs=[_row_spec, _col_spec],
      out_shape=[jax.ShapeDtypeStruct((N, H), jnp.float32),
                 jax.ShapeDtypeStruct((N, 1), jnp.float32)],
  )(h, wr, b, wc)


def _deg_body(hist, dinv_out):
  deg = jnp.sum(hist[:], axis=0)[:, None]
  dinv_out[:] = 1.0 / jnp.maximum(deg, 1.0)


def _deg(hist):
  return pl.pallas_call(
      _deg_body,
      grid=(1,),
      in_specs=[pl.BlockSpec((NW, N), lambda i: (0, 0))],
      out_specs=pl.BlockSpec((N, 1), lambda i: (0, 0)),
      out_shape=jax.ShapeDtypeStruct((N, 1), jnp.float32),
  )(hist)


def _tcb1_body(pA, pB, dinv, r, wl, h_out):
  agg = (pA[0] + pB[0]) * dinv[:]
  h_out[:] = jax.nn.relu(jnp.dot(agg, wl[:], **_DOT) + r[:])


def _tcb1(p, dinv, r, wl):
  return pl.pallas_call(
      _tcb1_body,
      grid=(N // BN,),
      in_specs=[_pA_spec(H), _pB_spec(H), _col_spec, _row_spec, _w_spec],
      out_specs=_row_spec,
      out_shape=jax.ShapeDtypeStruct((N, H), jnp.float32),
  )(p, p, dinv, r, wl)


def _tcb2_body(pA, pB, dinv, r, wl, wc3, c1, c2, bc, o):
  agg = (pA[0] + pB[0]) * dinv[:]
  h3 = jax.nn.relu(jnp.dot(agg, wl[:], **_DOT) + r[:])
  o[:] = jnp.dot(h3, wc3[:], **_DOT) + c1[:] + c2[:] + bc[0]


def _tcb2(p, dinv, r, wl, wc3, c1, c2, bc):
  return pl.pallas_call(
      _tcb2_body,
      grid=(N // BN,),
      in_specs=[_pA_spec(H), _pB_spec(H), _col_spec, _row_spec, _w_spec,
                _wc_spec, _col_spec, _col_spec,
                pl.BlockSpec((1,), lambda i: (0,))],
      out_specs=_col_spec,
      out_shape=jax.ShapeDtypeStruct((N, 1), jnp.float32),
  )(p, p, dinv, r, wl, wc3, c1, c2, bc)


def kernel(x, edge_index, W_l0, W_r0, b0, W_l1, W_r1, b1, W_l2, W_r2, b2,
           W_c, b_c):
  e = edge_index.reshape(2, NW, EPT)
  e_main = e[:, :, :NBF * B].reshape(2, NW, NBF, B)
  e_tail = e[:, :, NBF * B:]

  p0, hist = _sc_agg_0(x, e_main, e_tail)
  r0 = _tca0(x, W_r0, b0)
  dinv = _deg(hist)
  h1 = _tcb1(p0, dinv, r0, W_l0)

  p1 = _sc_agg_h(h1, e_main, e_tail)
  r1, c1 = _tca(h1, W_r1, b1, W_c[0:H])
  h2 = _tcb1(p1, dinv, r1, W_l1)

  p2 = _sc_agg_h(h2, e_main, e_tail)
  r2, c2 = _tca(h2, W_r2, b2, W_c[H:2 * H])
  out = _tcb2(p2, dinv, r2, W_l2, W_c[2 * H:3 * H], c1, c2, b_c)
  return out[:, 0]


# speedup vs baseline: 13.9882x; 1.0389x over previous
import functools

import jax
import jax.numpy as jnp
from jax import lax
from jax.experimental import pallas as pl
from jax.experimental.pallas import tpu as pltpu
from jax.experimental.pallas import tpu_sc as plsc

N = 10000
E = 320000
D = 128
H = 128

NC = 2
NS = 16
NW = NC * NS
EPT = E // NW
B = 128
NBF = EPT // B
TAIL = EPT - NBF * B

RPT = N // NS
RC = 125
RB = RPT // RC

def _make_sc_agg(W, with_hist):
  mesh = plsc.VectorSubcoreMesh(core_axis_name="c", subcore_axis_name="s")

  out_type = jax.ShapeDtypeStruct((NC, N, W), jnp.float32)
  if with_hist:
    out_type = (out_type, jax.ShapeDtypeStruct((NW, N), jnp.float32))

  @functools.partial(
      pl.kernel,
      out_type=out_type,
      mesh=mesh,
      scratch_types=([pltpu.VMEM((N,), jnp.float32)] if with_hist else []) + [
          pltpu.VMEM((4, B), jnp.int32),
          pltpu.VMEM((4, B), jnp.int32),
          pltpu.VMEM((B, W), jnp.float32),
          pltpu.VMEM((B, W), jnp.float32),
          pltpu.VMEM((TAIL,), jnp.int32),
          pltpu.VMEM((TAIL,), jnp.int32),
          pltpu.VMEM_SHARED((N, W), jnp.float32),
          pltpu.SemaphoreType.DMA,
          pltpu.SemaphoreType.DMA,
          pltpu.SemaphoreType.DMA,
          pltpu.SemaphoreType.DMA,
          pltpu.SemaphoreType.DMA,
          pltpu.SemaphoreType.DMA,
          pltpu.SemaphoreType.DMA,
      ],
      compiler_params=pltpu.CompilerParams(use_tc_tiling_on_sc=False,
                                           needs_layout_passes=False),
  )
  def sc_agg(*args):
    if with_hist:
      (h_hbm, e_hbm, out_hbm, hist_hbm, hist,
       srcr, dstr, buf0, buf1, srct, dstt, acc,
       g0, g1, i0, i1, i2, i3, it) = args
    else:
      (h_hbm, e_hbm, out_hbm,
       srcr, dstr, buf0, buf1, srct, dstt, acc,
       g0, g1, i0, i1, i2, i3, it) = args
    c = lax.axis_index("c")
    s = lax.axis_index("s")
    w = c * NS + s
    bufs = (buf0, buf1)
    gsem = (g0, g1)
    isem = (i0, i1, i2, i3)

    zeros16 = jnp.zeros((16,), jnp.float32)
    ones16 = jnp.ones((16,), jnp.float32)

    def hist_update(idx_row):
      if with_hist:
        for j in range(0, B, 16):
          plsc.addupdate_scatter(hist, [idx_row[pl.ds(j, 16)]], ones16)

    @pl.loop(0, B)
    def _(i):
      @pl.loop(0, W, step=16)
      def _(j):
        buf0[i, pl.ds(j, 16)] = zeros16

    @pl.loop(0, RB)
    def _(k):
      pltpu.sync_copy(buf0.at[pl.ds(0, RC)],
                      acc.at[pl.ds(s * RPT + k * RC, RC)])

    pltpu.sync_copy(e_hbm.at[0, w, pl.ds(0, B)], srcr.at[0])
    pltpu.sync_copy(e_hbm.at[1, w, pl.ds(0, B)], dstr.at[0])
    pltpu.sync_copy(e_hbm.at[0, w, pl.ds(B, B)], srcr.at[1])
    pltpu.sync_copy(e_hbm.at[1, w, pl.ds(B, B)], dstr.at[1])
    pltpu.async_copy(e_hbm.at[0, w, pl.ds(2 * B, B)], srcr.at[2], i2)
    pltpu.async_copy(e_hbm.at[1, w, pl.ds(2 * B, B)], dstr.at[2], i2)
    pltpu.async_copy(e_hbm.at[0, w, pl.ds(3 * B, B)], srcr.at[3], i3)
    pltpu.async_copy(e_hbm.at[1, w, pl.ds(3 * B, B)], dstr.at[3], i3)
    pltpu.async_copy(e_hbm.at[0, w, pl.ds(NBF * B, TAIL)], srct, it)
    pltpu.async_copy(e_hbm.at[1, w, pl.ds(NBF * B, TAIL)], dstt, it)
    pltpu.async_copy(h_hbm.at[srcr.at[0]], buf0, g0)
    pltpu.async_copy(h_hbm.at[srcr.at[1]], buf1, g1)

    if with_hist:
      @pl.loop(0, N, step=16)
      def _(i):
        hist[pl.ds(i, 16)] = zeros16

    plsc.subcore_barrier()

    @pl.loop(0, NBF - 2, step=4)
    def _(g):
      for k in range(4):
        p = k % 2
        k2 = (k + 2) % 4
        b = g + k
        pltpu.make_async_copy(h_hbm.at[srcr.at[k]], bufs[p], gsem[p]).wait()
        pltpu.sync_copy(bufs[p], acc.at[dstr.at[k]], add=True)
        hist_update(dstr.at[k])

        @pl.when(b + 2 < NBF)
        def _():
          pltpu.make_async_copy(e_hbm.at[0, w, pl.ds(0, B)], srcr.at[k2],
                                isem[k2]).wait()
          pltpu.make_async_copy(e_hbm.at[1, w, pl.ds(0, B)], dstr.at[k2],
                                isem[k2]).wait()
          pltpu.async_copy(h_hbm.at[srcr.at[k2]], bufs[p], gsem[p])

        @pl.when(b + 4 < NBF)
        def _():
          pltpu.async_copy(e_hbm.at[0, w, pl.ds((b + 4) * B, B)],
                           srcr.at[k], isem[k])
          pltpu.async_copy(e_hbm.at[1, w, pl.ds((b + 4) * B, B)],
                           dstr.at[k], isem[k])

    pltpu.make_async_copy(h_hbm.at[srcr.at[0]], buf0, g0).wait()
    pltpu.sync_copy(buf0, acc.at[dstr.at[0]], add=True)
    hist_update(dstr.at[0])
    pltpu.make_async_copy(e_hbm.at[0, w, pl.ds(0, TAIL)], srct, it).wait()
    pltpu.make_async_copy(e_hbm.at[1, w, pl.ds(0, TAIL)], dstt, it).wait()
    pltpu.async_copy(h_hbm.at[srct], buf0.at[pl.ds(0, TAIL)], g0)
    pltpu.make_async_copy(h_hbm.at[srcr.at[1]], buf1, g1).wait()
    pltpu.sync_copy(buf1, acc.at[dstr.at[1]], add=True)
    hist_update(dstr.at[1])
    pltpu.make_async_copy(h_hbm.at[srct], buf0.at[pl.ds(0, TAIL)], g0).wait()
    pltpu.sync_copy(buf0.at[pl.ds(0, TAIL)], acc.at[dstt], add=True)
    if with_hist:
      plsc.addupdate_scatter(hist, [dstt[...]], ones16)
      pltpu.sync_copy(hist, hist_hbm.at[w])

    plsc.subcore_barrier()

    @pl.loop(0, RB)
    def _(k):
      pltpu.sync_copy(acc.at[pl.ds(s * RPT + k * RC, RC)],
                      out_hbm.at[c, pl.ds(s * RPT + k * RC, RC)])

  return sc_agg


_sc_agg_0 = _make_sc_agg(H, with_hist=True)
_sc_agg_h = _make_sc_agg(H, with_hist=False)

BN = 2000

_DOT = dict(preferred_element_type=jnp.float32,
            precision=lax.Precision.DEFAULT)

_row_spec = pl.BlockSpec((BN, H), lambda i: (i, 0))
_col_spec = pl.BlockSpec((BN, 1), lambda i: (i, 0))
_w_spec = pl.BlockSpec((H, H), lambda i: (0, 0))
_b_spec = pl.BlockSpec((H,), lambda i: (0,))
_wc_spec = pl.BlockSpec((H, 1), lambda i: (0, 0))


def _pA_spec(W):
  return pl.BlockSpec((1, BN, W), lambda i: (0, i, 0))


def _pB_spec(W):
  return pl.BlockSpec((1, BN, W), lambda i: (1, i, 0))


def _tca0_body(h, wr, b, r):
  r[:] = jnp.dot(h[:], wr[:], **_DOT) + b[:][None, :]


def _tca0(h, wr, b):
  return pl.pallas_call(
      _tca0_body,
      grid=(N // BN,),
      in_specs=[_row_spec, _w_spec, _b_spec],
      out_specs=_row_spec,
      out_shape=jax.ShapeDtypeStruct((N, H), jnp.float32),
  )(h, wr, b)


def _tca_body(h, wr, b, wc, r, cpart):
  r[:] = jnp.dot(h[:], wr[:], **_DOT) + b[:][None, :]
  cpart[:] = jnp.dot(h[:], wc[:], **_DOT)


def _tca(h, wr, b, wc):
  return pl.pallas_call(
      _tca_body,
      grid=(N // BN,),
      in_specs=[_row_spec, _w_spec, _b_spec, _wc_spec],
      out_specs=[_row_spec, _col_spec],
      out_shape=[jax.ShapeDtypeStruct((N, H), jnp.float32),
                 jax.ShapeDtypeStruct((N, 1), jnp.float32)],
  )(h, wr, b, wc)


def _deg_body(hist, dinv_out):
  deg = jnp.sum(hist[:], axis=0)[:, None]
  dinv_out[:] = 1.0 / jnp.maximum(deg, 1.0)


def _deg(hist):
  return pl.pallas_call(
      _deg_body,
      grid=(1,),
      in_specs=[pl.BlockSpec((NW, N), lambda i: (0, 0))],
      out_specs=pl.BlockSpec((N, 1), lambda i: (0, 0)),
      out_shape=jax.ShapeDtypeStruct((N, 1), jnp.float32),
  )(hist)


def _tcb1_body(pA, pB, dinv, r, wl, h_out):
  agg = (pA[0] + pB[0]) * dinv[:]
  h_out[:] = jax.nn.relu(jnp.dot(agg, wl[:], **_DOT) + r[:])


def _tcb1(p, dinv, r, wl):
  return pl.pallas_call(
      _tcb1_body,
      grid=(N // BN,),
      in_specs=[_pA_spec(H), _pB_spec(H), _col_spec, _row_spec, _w_spec],
      out_specs=_row_spec,
      out_shape=jax.ShapeDtypeStruct((N, H), jnp.float32),
  )(p, p, dinv, r, wl)


def _tcb2_body(pA, pB, dinv, r, wl, wc3, c1, c2, bc, o):
  agg = (pA[0] + pB[0]) * dinv[:]
  h3 = jax.nn.relu(jnp.dot(agg, wl[:], **_DOT) + r[:])
  o[:] = jnp.dot(h3, wc3[:], **_DOT) + c1[:] + c2[:] + bc[0]


def _tcb2(p, dinv, r, wl, wc3, c1, c2, bc):
  return pl.pallas_call(
      _tcb2_body,
      grid=(N // BN,),
      in_specs=[_pA_spec(H), _pB_spec(H), _col_spec, _row_spec, _w_spec,
                _wc_spec, _col_spec, _col_spec,
                pl.BlockSpec((1,), lambda i: (0,))],
      out_specs=_col_spec,
      out_shape=jax.ShapeDtypeStruct((N, 1), jnp.float32),
  )(p, p, dinv, r, wl, wc3, c1, c2, bc)


def kernel(x, edge_index, W_l0, W_r0, b0, W_l1, W_r1, b1, W_l2, W_r2, b2,
           W_c, b_c):
  e = edge_index.reshape(2, NW, EPT)

  p0, hist = _sc_agg_0(x, e)
  r0 = _tca0(x, W_r0, b0)
  dinv = _deg(hist)
  h1 = _tcb1(p0, dinv, r0, W_l0)

  p1 = _sc_agg_h(h1, e)
  r1, c1 = _tca(h1, W_r1, b1, W_c[0:H])
  h2 = _tcb1(p1, dinv, r1, W_l1)

  p2 = _sc_agg_h(h2, e)
  r2, c2 = _tca(h2, W_r2, b2, W_c[H:2 * H])
  out = _tcb2(p2, dinv, r2, W_l2, W_c[2 * H:3 * H], c1, c2, b_c)
  return jnp.reshape(out, (N,))


# speedup vs baseline: 13.9990x; 1.0008x over previous
import functools

import jax
import jax.numpy as jnp
from jax import lax
from jax.experimental import pallas as pl
from jax.experimental.pallas import tpu as pltpu
from jax.experimental.pallas import tpu_sc as plsc

N = 10000
E = 320000
D = 128
H = 128

NC = 2
NS = 16
NW = NC * NS
EPT = E // NW
B = 128
NBF = EPT // B
TAIL = EPT - NBF * B

RPT = N // NS
RC = 125
RB = RPT // RC

def _make_sc_agg(W, with_hist):
  mesh = plsc.VectorSubcoreMesh(core_axis_name="c", subcore_axis_name="s")

  out_type = jax.ShapeDtypeStruct((NC, N, W), jnp.float32)
  if with_hist:
    out_type = (out_type, jax.ShapeDtypeStruct((NW, N), jnp.float32))

  @functools.partial(
      pl.kernel,
      out_type=out_type,
      mesh=mesh,
      scratch_types=([pltpu.VMEM((N,), jnp.float32)] if with_hist else []) + [
          pltpu.VMEM((4, B), jnp.int32),
          pltpu.VMEM((4, B), jnp.int32),
          pltpu.VMEM((B, W), jnp.float32),
          pltpu.VMEM((B, W), jnp.float32),
          pltpu.VMEM((TAIL,), jnp.int32),
          pltpu.VMEM((TAIL,), jnp.int32),
          pltpu.VMEM_SHARED((N, W), jnp.float32),
          pltpu.SemaphoreType.DMA,
          pltpu.SemaphoreType.DMA,
          pltpu.SemaphoreType.DMA,
          pltpu.SemaphoreType.DMA,
          pltpu.SemaphoreType.DMA,
          pltpu.SemaphoreType.DMA,
          pltpu.SemaphoreType.DMA,
          pltpu.SemaphoreType.DMA,
      ],
      compiler_params=pltpu.CompilerParams(use_tc_tiling_on_sc=False,
                                           needs_layout_passes=False),
  )
  def sc_agg(*args):
    if with_hist:
      (h_hbm, e_hbm, out_hbm, hist_hbm, hist,
       srcr, dstr, buf0, buf1, srct, dstt, acc,
       g0, g1, i0, i1, i2, i3, it, z) = args
    else:
      (h_hbm, e_hbm, out_hbm,
       srcr, dstr, buf0, buf1, srct, dstt, acc,
       g0, g1, i0, i1, i2, i3, it, z) = args
    c = lax.axis_index("c")
    s = lax.axis_index("s")
    w = c * NS + s
    bufs = (buf0, buf1)
    gsem = (g0, g1)
    isem = (i0, i1, i2, i3)

    zeros16 = jnp.zeros((16,), jnp.float32)
    ones16 = jnp.ones((16,), jnp.float32)

    def hist_update(idx_row):
      if with_hist:
        for j in range(0, B, 16):
          plsc.addupdate_scatter(hist, [idx_row[pl.ds(j, 16)]], ones16)

    pltpu.sync_copy(e_hbm.at[0, w, pl.ds(0, B)], srcr.at[0])
    pltpu.sync_copy(e_hbm.at[1, w, pl.ds(0, B)], dstr.at[0])
    pltpu.sync_copy(e_hbm.at[0, w, pl.ds(B, B)], srcr.at[1])
    pltpu.sync_copy(e_hbm.at[1, w, pl.ds(B, B)], dstr.at[1])
    pltpu.async_copy(e_hbm.at[0, w, pl.ds(2 * B, B)], srcr.at[2], i2)
    pltpu.async_copy(e_hbm.at[1, w, pl.ds(2 * B, B)], dstr.at[2], i2)
    pltpu.async_copy(e_hbm.at[0, w, pl.ds(3 * B, B)], srcr.at[3], i3)
    pltpu.async_copy(e_hbm.at[1, w, pl.ds(3 * B, B)], dstr.at[3], i3)
    pltpu.async_copy(e_hbm.at[0, w, pl.ds(NBF * B, TAIL)], srct, it)
    pltpu.async_copy(e_hbm.at[1, w, pl.ds(NBF * B, TAIL)], dstt, it)

    pltpu.async_copy(h_hbm.at[srcr.at[1]], buf1, g1)

    @pl.loop(0, B)
    def _(i):
      @pl.loop(0, W, step=16)
      def _(j):
        buf0[i, pl.ds(j, 16)] = zeros16

    for k in range(RB):
      pltpu.async_copy(buf0.at[pl.ds(0, RC)],
                       acc.at[pl.ds(s * RPT + k * RC, RC)], z)
    for k in range(RB):
      pltpu.make_async_copy(buf0.at[pl.ds(0, RC)],
                            acc.at[pl.ds(s * RPT + k * RC, RC)], z).wait()
    pltpu.async_copy(h_hbm.at[srcr.at[0]], buf0, g0)

    if with_hist:
      @pl.loop(0, N, step=16)
      def _(i):
        hist[pl.ds(i, 16)] = zeros16

    plsc.subcore_barrier()

    @pl.loop(0, NBF - 2, step=4)
    def _(g):
      for k in range(4):
        p = k % 2
        k2 = (k + 2) % 4
        b = g + k
        pltpu.make_async_copy(h_hbm.at[srcr.at[k]], bufs[p], gsem[p]).wait()
        pltpu.sync_copy(bufs[p], acc.at[dstr.at[k]], add=True)
        hist_update(dstr.at[k])

        @pl.when(b + 2 < NBF)
        def _():
          pltpu.make_async_copy(e_hbm.at[0, w, pl.ds(0, B)], srcr.at[k2],
                                isem[k2]).wait()
          pltpu.make_async_copy(e_hbm.at[1, w, pl.ds(0, B)], dstr.at[k2],
                                isem[k2]).wait()
          pltpu.async_copy(h_hbm.at[srcr.at[k2]], bufs[p], gsem[p])

        @pl.when(b + 4 < NBF)
        def _():
          pltpu.async_copy(e_hbm.at[0, w, pl.ds((b + 4) * B, B)],
                           srcr.at[k], isem[k])
          pltpu.async_copy(e_hbm.at[1, w, pl.ds((b + 4) * B, B)],
                           dstr.at[k], isem[k])

    pltpu.make_async_copy(h_hbm.at[srcr.at[0]], buf0, g0).wait()
    pltpu.sync_copy(buf0, acc.at[dstr.at[0]], add=True)
    hist_update(dstr.at[0])
    pltpu.make_async_copy(e_hbm.at[0, w, pl.ds(0, TAIL)], srct, it).wait()
    pltpu.make_async_copy(e_hbm.at[1, w, pl.ds(0, TAIL)], dstt, it).wait()
    pltpu.async_copy(h_hbm.at[srct], buf0.at[pl.ds(0, TAIL)], g0)
    pltpu.make_async_copy(h_hbm.at[srcr.at[1]], buf1, g1).wait()
    pltpu.sync_copy(buf1, acc.at[dstr.at[1]], add=True)
    hist_update(dstr.at[1])
    pltpu.make_async_copy(h_hbm.at[srct], buf0.at[pl.ds(0, TAIL)], g0).wait()
    pltpu.sync_copy(buf0.at[pl.ds(0, TAIL)], acc.at[dstt], add=True)
    if with_hist:
      plsc.addupdate_scatter(hist, [dstt[...]], ones16)
      pltpu.sync_copy(hist, hist_hbm.at[w])

    plsc.subcore_barrier()

    for k in range(RB):
      pltpu.async_copy(acc.at[pl.ds(s * RPT + k * RC, RC)],
                       out_hbm.at[c, pl.ds(s * RPT + k * RC, RC)], z)
    for k in range(RB):
      pltpu.make_async_copy(acc.at[pl.ds(s * RPT + k * RC, RC)],
                            out_hbm.at[c, pl.ds(s * RPT + k * RC, RC)],
                            z).wait()

  return sc_agg


_sc_agg_0 = _make_sc_agg(H, with_hist=True)
_sc_agg_h = _make_sc_agg(H, with_hist=False)

BN = 2000

_DOT = dict(preferred_element_type=jnp.float32,
            precision=lax.Precision.DEFAULT)

_row_spec = pl.BlockSpec((BN, H), lambda i: (i, 0))
_col_spec = pl.BlockSpec((BN, 1), lambda i: (i, 0))
_w_spec = pl.BlockSpec((H, H), lambda i: (0, 0))
_b_spec = pl.BlockSpec((H,), lambda i: (0,))
_wc_spec = pl.BlockSpec((H, 1), lambda i: (0, 0))


def _pA_spec(W):
  return pl.BlockSpec((1, BN, W), lambda i: (0, i, 0))


def _pB_spec(W):
  return pl.BlockSpec((1, BN, W), lambda i: (1, i, 0))


def _tca0_body(h, wr, b, r):
  r[:] = jnp.dot(h[:], wr[:], **_DOT) + b[:][None, :]


def _tca0(h, wr, b):
  return pl.pallas_call(
      _tca0_body,
      grid=(N // BN,),
      in_specs=[_row_spec, _w_spec, _b_spec],
      out_specs=_row_spec,
      out_shape=jax.ShapeDtypeStruct((N, H), jnp.float32),
  )(h, wr, b)


def _tca_body(h, wr, b, wc, r, cpart):
  r[:] = jnp.dot(h[:], wr[:], **_DOT) + b[:][None, :]
  cpart[:] = jnp.dot(h[:], wc[:], **_DOT)


def _tca(h, wr, b, wc):
  return pl.pallas_call(
      _tca_body,
      grid=(N // BN,),
      in_specs=[_row_spec, _w_spec, _b_spec, _wc_spec],
      out_specs=[_row_spec, _col_spec],
      out_shape=[jax.ShapeDtypeStruct((N, H), jnp.float32),
                 jax.ShapeDtypeStruct((N, 1), jnp.float32)],
  )(h, wr, b, wc)


def _deg_body(hist, dinv_out):
  deg = jnp.sum(hist[:], axis=0)[:, None]
  dinv_out[:] = 1.0 / jnp.maximum(deg, 1.0)


def _deg(hist):
  return pl.pallas_call(
      _deg_body,
      grid=(1,),
      in_specs=[pl.BlockSpec((NW, N), lambda i: (0, 0))],
      out_specs=pl.BlockSpec((N, 1), lambda i: (0, 0)),
      out_shape=jax.ShapeDtypeStruct((N, 1), jnp.float32),
  )(hist)


def _tcb1_body(pA, pB, dinv, r, wl, h_out):
  agg = (pA[0] + pB[0]) * dinv[:]
  h_out[:] = jax.nn.relu(jnp.dot(agg, wl[:], **_DOT) + r[:])


def _tcb1(p, dinv, r, wl):
  return pl.pallas_call(
      _tcb1_body,
      grid=(N // BN,),
      in_specs=[_pA_spec(H), _pB_spec(H), _col_spec, _row_spec, _w_spec],
      out_specs=_row_spec,
      out_shape=jax.ShapeDtypeStruct((N, H), jnp.float32),
  )(p, p, dinv, r, wl)


def _tcb2_body(pA, pB, dinv, r, wl, wc3, c1, c2, bc, o):
  agg = (pA[0] + pB[0]) * dinv[:]
  h3 = jax.nn.relu(jnp.dot(agg, wl[:], **_DOT) + r[:])
  o[:] = jnp.dot(h3, wc3[:], **_DOT) + c1[:] + c2[:] + bc[0]


def _tcb2(p, dinv, r, wl, wc3, c1, c2, bc):
  return pl.pallas_call(
      _tcb2_body,
      grid=(N // BN,),
      in_specs=[_pA_spec(H), _pB_spec(H), _col_spec, _row_spec, _w_spec,
                _wc_spec, _col_spec, _col_spec,
                pl.BlockSpec((1,), lambda i: (0,))],
      out_specs=_col_spec,
      out_shape=jax.ShapeDtypeStruct((N, 1), jnp.float32),
  )(p, p, dinv, r, wl, wc3, c1, c2, bc)


def kernel(x, edge_index, W_l0, W_r0, b0, W_l1, W_r1, b1, W_l2, W_r2, b2,
           W_c, b_c):
  e = edge_index.reshape(2, NW, EPT)

  p0, hist = _sc_agg_0(x, e)
  r0 = _tca0(x, W_r0, b0)
  dinv = _deg(hist)
  h1 = _tcb1(p0, dinv, r0, W_l0)

  p1 = _sc_agg_h(h1, e)
  r1, c1 = _tca(h1, W_r1, b1, W_c[0:H])
  h2 = _tcb1(p1, dinv, r1, W_l1)

  p2 = _sc_agg_h(h2, e)
  r2, c2 = _tca(h2, W_r2, b2, W_c[H:2 * H])
  out = _tcb2(p2, dinv, r2, W_l2, W_c[2 * H:3 * H], c1, c2, b_c)
  return jnp.reshape(out, (N,))
